# Initial kernel scaffold; baseline (speedup 1.0000x reference)
import jax
import jax.numpy as jnp
from jax.experimental import pallas as pl


def kernel(source_node, target_node, edge_index, edge_attr, distance, W_msg, b_msg, W_res, W_comb, b_comb, ln_gamma, ln_beta):
    raise NotImplementedError("write your pallas kernel here")


# speedup vs baseline: 1.1410x; 1.1410x over previous
import functools

import jax
import jax.numpy as jnp
from jax import lax
from jax.experimental import pallas as pl
from jax.experimental.pallas import tpu as pltpu
from jax.experimental.pallas import tpu_sc as plsc

N = 10000
E = 320000
D = 128
H = 128
OUT = 128
HD = 64

NC = 2
NS = 16
L = 16
EPS = E // NS
C = 80
NCHUNK = EPS // C
AROWS = 5120
ARPS = AROWS // NS
P = 64
CROWS = 160
CSUB = 5
CRPS = CROWS // CSUB
ZROWS = 128


def _proj_body(src_ref, tgt_ref, wst_ref, wtt_ref, b_ref, ps_ref, pt_ref):
    ps_ref[...] = lax.dot_general(
        src_ref[...], wst_ref[...], (((1,), (0,)), ((), ())),
        precision=lax.Precision.HIGHEST, preferred_element_type=jnp.float32)
    pt_ref[...] = lax.dot_general(
        tgt_ref[...], wtt_ref[...], (((1,), (0,)), ((), ())),
        precision=lax.Precision.HIGHEST, preferred_element_type=jnp.float32) + b_ref[...]


def _project_tables(src, tgt, wst, wtt, b):
    bn = 2000
    grid = (N // bn,)
    return pl.pallas_call(
        _proj_body,
        grid=grid,
        in_specs=[
            pl.BlockSpec((bn, D), lambda i: (i, 0)),
            pl.BlockSpec((bn, D), lambda i: (i, 0)),
            pl.BlockSpec((D, H), lambda i: (0, 0)),
            pl.BlockSpec((D, H), lambda i: (0, 0)),
            pl.BlockSpec((1, H), lambda i: (0, 0)),
        ],
        out_specs=[
            pl.BlockSpec((bn, H), lambda i: (i, 0)),
            pl.BlockSpec((bn, H), lambda i: (i, 0)),
        ],
        out_shape=[
            jax.ShapeDtypeStruct((N, H), jnp.float32),
            jax.ShapeDtypeStruct((N, H), jnp.float32),
        ],
    )(src, tgt, wst, wtt, b)


def _edge_body(ps_hbm, pt_hbm, is_hbm, it_hbm, dist_hbm, wd_hbm,
               msg_hbm, cnt_hbm,
               isv, itv, itp, ith, itcv, dv, psv, ptv, mv, cv, zv,
               wdv, acc, cacc, sem1, sem2):
    cid = lax.axis_index("c")
    sid = lax.axis_index("s")

    def zrow(r, carry):
        for v in range(D // L):
            zv[r, pl.ds(v * L, L)] = jnp.zeros((L,), jnp.float32)
        return carry
    lax.fori_loop(0, ZROWS, zrow, 0)
    zoff = 0
    for rows in (ZROWS, ZROWS, ARPS - 2 * ZROWS):
        pltpu.sync_copy(zv.at[pl.ds(0, rows)],
                        acc.at[pl.ds(sid * ARPS + zoff, rows)])
        zoff += rows

    @pl.when(sid < CSUB)
    def _zero_counts():
        pltpu.sync_copy(zv.at[pl.ds(0, CRPS)],
                        cacc.at[pl.ds(sid * CRPS, CRPS)])

    def czrow(r, carry):
        for v in range(P // L, D // L):
            cv[r, pl.ds(v * L, L)] = jnp.zeros((L,), jnp.float32)
        return carry
    lax.fori_loop(0, C, czrow, 0)

    pltpu.sync_copy(wd_hbm, wdv)
    plsc.subcore_barrier()

    wd_chunks = [wdv[pl.ds(cid * HD + v * L, L)] for v in range(HD // L)]
    lane_ids = lax.iota(jnp.int32, L)
    base0 = sid * EPS
    hoff = cid * HD

    def chunk_body(k, carry):
        base = base0 + k * C
        pltpu.sync_copy(is_hbm.at[pl.ds(base, C)], isv)
        pltpu.sync_copy(it_hbm.at[pl.ds(base, C)], itv)
        pltpu.sync_copy(it_hbm.at[pl.ds(base, C)], itp.at[pl.ds(0, C)])
        pltpu.sync_copy(dist_hbm.at[pl.ds(base, C)], dv.at[pl.ds(0, C)])

        def gidx(g, carry2):
            sl = pl.ds(g * L, L)
            ith[sl] = lax.shift_right_logical(itv[sl], 1)
            return carry2
        lax.fori_loop(0, C // L, gidx, 0)

        cp1 = pltpu.async_copy(ps_hbm.at[isv], psv, sem1)
        cp2 = pltpu.async_copy(pt_hbm.at[itv], ptv, sem2)
        cp1.wait()
        cp2.wait()

        def row_body(r, rc):
            dist = dv[pl.ds(r, L)][0]
            it_s = itp[pl.ds(r, L)][0]
            m0 = ((it_s & 1) == 0).astype(jnp.float32)
            m1 = 1.0 - m0
            for v in range(HD // L):
                x = psv[r, pl.ds(hoff + v * L, L)] \
                    + ptv[r, pl.ds(hoff + v * L, L)] \
                    + dist * wd_chunks[v]
                m = jnp.where(x > 0, x, jnp.exp(x) - 1.0)
                mv[r, pl.ds(v * L, L)] = m * m0
                mv[r, pl.ds(HD + v * L, L)] = m * m1
            return rc
        lax.fori_loop(0, C, row_body, 0)

        pltpu.sync_copy(mv, acc.at[ith], add=True)

        @pl.when((k & 1) == cid)
        def _count_chunk():
            def cidx(g, carry2):
                itcv[pl.ds(g * L, L)] = lax.shift_right_logical(
                    itv[pl.ds(g * L, L)], 6)
                return carry2
            lax.fori_loop(0, C // L, cidx, 0)

            def crow(r, rc):
                it_s = itp[pl.ds(r, L)][0]
                eq = lane_ids == (it_s & 15)
                ch = lax.shift_right_logical(it_s, 4) & 3
                for v in range(P // L):
                    csf = (ch == v).astype(jnp.float32)
                    cv[r, pl.ds(v * L, L)] = jnp.where(eq, csf, 0.0)
                return rc
            lax.fori_loop(0, C, crow, 0)
            pltpu.sync_copy(cv, cacc.at[itcv], add=True)
        return carry
    lax.fori_loop(0, NCHUNK, chunk_body, 0)

    plsc.subcore_barrier()
    pltpu.sync_copy(acc.at[pl.ds(sid * ARPS, ARPS)],
                    msg_hbm.at[cid, pl.ds(sid * ARPS, ARPS)])

    @pl.when(sid < CSUB)
    def _flush_counts():
        pltpu.sync_copy(cacc.at[pl.ds(sid * CRPS, CRPS)],
                        cnt_hbm.at[cid, pl.ds(sid * CRPS, CRPS)])


_edge_kernel = functools.partial(
    pl.kernel,
    out_type=[
        jax.ShapeDtypeStruct((NC, AROWS, D), jnp.float32),
        jax.ShapeDtypeStruct((NC, CROWS, D), jnp.float32),
    ],
    mesh=plsc.VectorSubcoreMesh(core_axis_name="c", subcore_axis_name="s"),
    scratch_types=[
        pltpu.VMEM((C,), jnp.int32),
        pltpu.VMEM((C,), jnp.int32),
        pltpu.VMEM((C + L,), jnp.int32),
        pltpu.VMEM((C,), jnp.int32),
        pltpu.VMEM((C,), jnp.int32),
        pltpu.VMEM((C + L,), jnp.float32),
        pltpu.VMEM((C, D), jnp.float32),
        pltpu.VMEM((C, D), jnp.float32),
        pltpu.VMEM((C, D), jnp.float32),
        pltpu.VMEM((C, D), jnp.float32),
        pltpu.VMEM((ZROWS, D), jnp.float32),
        pltpu.VMEM((D,), jnp.float32),
        pltpu.VMEM_SHARED((AROWS, D), jnp.float32),
        pltpu.VMEM_SHARED((CROWS, D), jnp.float32),
        pltpu.SemaphoreType.DMA,
        pltpu.SemaphoreType.DMA,
    ],
)(_edge_body)


def _node_body(s0_ref, s1_ref, cnt_ref, tgt_ref, wxt_ref, wa0_ref, wa1_ref,
               b_ref, g_ref, bt_ref, out_ref):
    cnt = jnp.maximum(cnt_ref[:, 0:1] + cnt_ref[:, 1:2], 1.0)
    y = lax.dot_general(
        tgt_ref[...], wxt_ref[...], (((1,), (0,)), ((), ())),
        precision=lax.Precision.HIGHEST, preferred_element_type=jnp.float32)
    y += lax.dot_general(
        s0_ref[...] / cnt, wa0_ref[...], (((1,), (0,)), ((), ())),
        precision=lax.Precision.HIGHEST, preferred_element_type=jnp.float32)
    y += lax.dot_general(
        s1_ref[...] / cnt, wa1_ref[...], (((1,), (0,)), ((), ())),
        precision=lax.Precision.HIGHEST, preferred_element_type=jnp.float32)
    y += b_ref[...]
    y = jnp.where(y > 0, y, jnp.exp(y) - 1.0)
    mean = jnp.mean(y, axis=-1, keepdims=True)
    yc = y - mean
    var = jnp.mean(yc * yc, axis=-1, keepdims=True)
    out_ref[...] = yc * lax.rsqrt(var + 1e-5) * g_ref[...] + bt_ref[...]


def _node_finalize(s0, s1, cnt_t, tgt, wxt, wa0, wa1, b, g, bt):
    bn = 2000
    grid = (N // bn,)
    return pl.pallas_call(
        _node_body,
        grid=grid,
        in_specs=[
            pl.BlockSpec((bn, HD), lambda i: (i, 0)),
            pl.BlockSpec((bn, HD), lambda i: (i, 0)),
            pl.BlockSpec((bn, NC), lambda i: (i, 0)),
            pl.BlockSpec((bn, D), lambda i: (i, 0)),
            pl.BlockSpec((D, OUT), lambda i: (0, 0)),
            pl.BlockSpec((HD, OUT), lambda i: (0, 0)),
            pl.BlockSpec((HD, OUT), lambda i: (0, 0)),
            pl.BlockSpec((1, OUT), lambda i: (0, 0)),
            pl.BlockSpec((1, OUT), lambda i: (0, 0)),
            pl.BlockSpec((1, OUT), lambda i: (0, 0)),
        ],
        out_specs=pl.BlockSpec((bn, OUT), lambda i: (i, 0)),
        out_shape=jax.ShapeDtypeStruct((N, OUT), jnp.float32),
    )(s0, s1, cnt_t, tgt, wxt, wa0, wa1, b, g, bt)


def kernel(source_node, target_node, edge_index, edge_attr, distance,
           W_msg, b_msg, W_res, W_comb, b_comb, ln_gamma, ln_beta):
    del edge_attr
    wst = W_msg[:, :D].T
    wtt = W_msg[:, D:2 * D].T
    wd = W_msg[:, 2 * D]
    ps, pt = _project_tables(source_node, target_node, wst, wtt,
                             b_msg.reshape(1, H))
    i_source = edge_index[0]
    i_target = edge_index[1]
    sums, cnt_packed = _edge_kernel(ps, pt, i_source, i_target,
                                    distance.reshape(E), wd)
    s0 = sums[0].reshape(2 * AROWS, HD)
    s1 = sums[1].reshape(2 * AROWS, HD)
    cnt_t = cnt_packed[:, :, :P].reshape(NC, CROWS * P).T
    wxt = (W_res + W_comb[:, :D]).T
    wat = W_comb[:, D:].T
    return _node_finalize(s0, s1, cnt_t, target_node, wxt,
                          wat[:HD], wat[HD:],
                          b_comb.reshape(1, OUT),
                          ln_gamma.reshape(1, OUT), ln_beta.reshape(1, OUT))


# speedup vs baseline: 4.0551x; 3.5538x over previous
import functools

import jax
import jax.numpy as jnp
from jax import lax
from jax.experimental import pallas as pl
from jax.experimental.pallas import tpu as pltpu
from jax.experimental.pallas import tpu_sc as plsc

N = 10000
E = 320000
D = 128
H = 128
OUT = 128
HD = 64

NC = 2
NS = 16
L = 16
EPS = E // NS
C = 80
NCHUNK = EPS // C
AROWS = 5120
ARPS = AROWS // NS
P = 64
CROWS = 160
CSUB = 5
CRPS = CROWS // CSUB
ZROWS = 64
PKW = 96


def _proj_body(src_ref, tgt_ref, wst_ref, wtt_ref, b_ref, ps_ref, pt_ref):
    ps_ref[...] = lax.dot_general(
        src_ref[...], wst_ref[...], (((1,), (0,)), ((), ())),
        precision=lax.Precision.HIGHEST, preferred_element_type=jnp.float32)
    pt_ref[...] = lax.dot_general(
        tgt_ref[...], wtt_ref[...], (((1,), (0,)), ((), ())),
        precision=lax.Precision.HIGHEST, preferred_element_type=jnp.float32) + b_ref[...]


def _project_tables(src, tgt, wst, wtt, b):
    bn = 2000
    grid = (N // bn,)
    return pl.pallas_call(
        _proj_body,
        grid=grid,
        in_specs=[
            pl.BlockSpec((bn, D), lambda i: (i, 0)),
            pl.BlockSpec((bn, D), lambda i: (i, 0)),
            pl.BlockSpec((D, H), lambda i: (0, 0)),
            pl.BlockSpec((D, H), lambda i: (0, 0)),
            pl.BlockSpec((1, H), lambda i: (0, 0)),
        ],
        out_specs=[
            pl.BlockSpec((bn, H), lambda i: (i, 0)),
            pl.BlockSpec((bn, H), lambda i: (i, 0)),
        ],
        out_shape=[
            jax.ShapeDtypeStruct((N, H), jnp.float32),
            jax.ShapeDtypeStruct((N, H), jnp.float32),
        ],
    )(src, tgt, wst, wtt, b)


def _bcast(v16, j):
    return v16.at[jnp.full((L,), j, jnp.int32)].get(mode="promise_in_bounds")


def _bcast_i(v16, j):
    return v16.at[jnp.full((L,), j, jnp.int32)].get(mode="promise_in_bounds")


def _edge_body(ps_hbm, pt_hbm, pk_hbm, wd_hbm,
               msg_hbm, cnt_hbm,
               pkv0, pkv1, ith0, ith1, itcv0, itcv1,
               psv0, psv1, ptv0, ptv1, mv0, mv1, cv0, zv, wdv,
               acc, cacc,
               spk0, spk1, sg0, sg1, ss0, ss1):
    cid = lax.axis_index("c")
    sid = lax.axis_index("s")

    def zrow(r, carry):
        for v in range(D // L):
            zv[r, pl.ds(v * L, L)] = jnp.zeros((L,), jnp.float32)
        return carry
    lax.fori_loop(0, ZROWS, zrow, 0)
    for b in range(ARPS // ZROWS):
        pltpu.sync_copy(zv, acc.at[pl.ds(sid * ARPS + b * ZROWS, ZROWS)])

    @pl.when(sid < CSUB)
    def _zero_counts():
        pltpu.sync_copy(zv.at[pl.ds(0, CRPS)],
                        cacc.at[pl.ds(sid * CRPS, CRPS)])

    def czrow(r, carry):
        for v in range(P // L, D // L):
            cv0[r, pl.ds(v * L, L)] = jnp.zeros((L,), jnp.float32)
        return carry
    lax.fori_loop(0, C, czrow, 0)

    pltpu.sync_copy(wd_hbm, wdv)
    plsc.subcore_barrier()

    hoff = cid * HD
    wd_chunks = [wdv[pl.ds(hoff + v * L, L)] for v in range(HD // L)]
    lane_ids = lax.iota(jnp.int32, L)
    ck0 = sid * NCHUNK

    pkv = (pkv0, pkv1)
    ith = (ith0, ith1)
    itcv = (itcv0, itcv1)
    psv = (psv0, psv1)
    ptv = (ptv0, ptv1)
    mv = (mv0, mv1)
    spk = (spk0, spk1)
    sg = (sg0, sg1)
    ss = (ss0, ss1)

    def start_pk(k, slot):
        pltpu.async_copy(pk_hbm.at[ck0 + k], pkv[slot], spk[slot])

    def wait_pk(k, slot):
        pltpu.make_async_copy(pk_hbm.at[ck0 + k], pkv[slot],
                              spk[slot]).wait()

    def gather_idx(slot, row):
        return pkv[slot].at[row, pl.ds(0, C)]

    def start_gathers(slot):
        pltpu.async_copy(ps_hbm.at[gather_idx(slot, 0)], psv[slot], sg[slot])
        pltpu.async_copy(pt_hbm.at[gather_idx(slot, 1)], ptv[slot], sg[slot])

    def wait_gathers(slot):
        pltpu.make_async_copy(ps_hbm.at[gather_idx(slot, 0)], psv[slot],
                              sg[slot]).wait()
        pltpu.make_async_copy(pt_hbm.at[gather_idx(slot, 1)], ptv[slot],
                              sg[slot]).wait()

    def wait_msg_scatter(slot):
        pltpu.make_async_copy(mv[slot], acc.at[ith[slot]], ss[slot]).wait()

    pltpu.sync_copy(pk_hbm.at[ck0], pkv0)
    start_gathers(0)
    start_pk(1, 1)

    def outer_body(k0, carry):
        for b in range(2):
            k = 2 * k0 + b
            cur, nxt = b, 1 - b

            @pl.when(k + 1 < NCHUNK)
            def _launch_next():
                wait_pk(k + 1, nxt)
                start_gathers(nxt)

            @pl.when(k >= 2)
            def _reclaim():
                wait_msg_scatter(cur)

            pkc = pkv[cur]

            def idx_body(g, carry2):
                it16 = pkc[1, pl.ds(g * L, L)]
                ith[cur][pl.ds(g * L, L)] = lax.shift_right_logical(it16, 1)
                itcv[cur][pl.ds(g * L, L)] = lax.shift_right_logical(it16, 6)
                return carry2
            lax.fori_loop(0, C // L, idx_body, 0)

            wait_gathers(cur)

            psc, ptc, mvc = psv[cur], ptv[cur], mv[cur]

            @plsc.parallel_loop(0, C // L, unroll=2)
            def group_body(g):
                g0 = g * L
                it16 = pkc[1, pl.ds(g0, L)]
                d16 = pkc[2, pl.ds(g0, L)].astype(jnp.float32) \
                    * 5.9604644775390625e-08
                m16 = jnp.where((it16 & 1) == 0, 1.0, 0.0)
                for j in range(0, L, 8):
                    rs = tuple(g0 + j + e for e in range(8))
                    dists = tuple(_bcast(d16, j + e) for e in range(8))
                    m0s = tuple(_bcast(m16, j + e) for e in range(8))
                    xs = [psc[r, pl.ds(hoff + v * L, L)]
                          + ptc[r, pl.ds(hoff + v * L, L)]
                          + dists[e] * wd_chunks[v]
                          for e, r in enumerate(rs)
                          for v in range(HD // L)]
                    es = [jnp.exp(x) for x in xs]
                    for e, r in enumerate(rs):
                        for v in range(HD // L):
                            i = e * (HD // L) + v
                            m = jnp.where(xs[i] > 0, xs[i], es[i] - 1.0)
                            lo = m * m0s[e]
                            mvc[r, pl.ds(v * L, L)] = lo
                            mvc[r, pl.ds(HD + v * L, L)] = m - lo

            pltpu.async_copy(mvc, acc.at[ith[cur]], ss[cur], add=True)

            @pl.when(cid == b)
            def _count_chunk():
                cvc = cv0

                @plsc.parallel_loop(0, C // L, unroll=2)
                def cgroup(g):
                    g0 = g * L
                    l16 = pkc[1, pl.ds(g0, L)] & 63
                    for j in range(L):
                        r = g0 + j
                        lv = _bcast_i(l16, j)
                        for v in range(P // L):
                            cvc[r, pl.ds(v * L, L)] = jnp.where(
                                lane_ids == lv - (v * L), 1.0, 0.0)
                pltpu.sync_copy(cvc, cacc.at[itcv[cur]], add=True)

            @pl.when(k + 2 < NCHUNK)
            def _prefetch():
                start_pk(k + 2, cur)
        return carry
    lax.fori_loop(0, NCHUNK // 2, outer_body, 0)

    for b in range(2):
        wait_msg_scatter(b)

    plsc.subcore_barrier()
    pltpu.sync_copy(acc.at[pl.ds(sid * ARPS, ARPS)],
                    msg_hbm.at[cid, pl.ds(sid * ARPS, ARPS)])

    @pl.when(sid < CSUB)
    def _flush_counts():
        pltpu.sync_copy(cacc.at[pl.ds(sid * CRPS, CRPS)],
                        cnt_hbm.at[cid, pl.ds(sid * CRPS, CRPS)])


_edge_kernel = functools.partial(
    pl.kernel,
    out_type=[
        jax.ShapeDtypeStruct((NC, AROWS, D), jnp.float32),
        jax.ShapeDtypeStruct((NC, CROWS, D), jnp.float32),
    ],
    mesh=plsc.VectorSubcoreMesh(core_axis_name="c", subcore_axis_name="s"),
    scratch_types=[
        pltpu.VMEM((4, PKW), jnp.int32),
        pltpu.VMEM((4, PKW), jnp.int32),
        pltpu.VMEM((C,), jnp.int32),
        pltpu.VMEM((C,), jnp.int32),
        pltpu.VMEM((C,), jnp.int32),
        pltpu.VMEM((C,), jnp.int32),
        pltpu.VMEM((C, D), jnp.float32),
        pltpu.VMEM((C, D), jnp.float32),
        pltpu.VMEM((C, D), jnp.float32),
        pltpu.VMEM((C, D), jnp.float32),
        pltpu.VMEM((C, D), jnp.float32),
        pltpu.VMEM((C, D), jnp.float32),
        pltpu.VMEM((C, D), jnp.float32),
        pltpu.VMEM((ZROWS, D), jnp.float32),
        pltpu.VMEM((D,), jnp.float32),
        pltpu.VMEM_SHARED((AROWS, D), jnp.float32),
        pltpu.VMEM_SHARED((CROWS, D), jnp.float32),
        pltpu.SemaphoreType.DMA,
        pltpu.SemaphoreType.DMA,
        pltpu.SemaphoreType.DMA,
        pltpu.SemaphoreType.DMA,
        pltpu.SemaphoreType.DMA,
        pltpu.SemaphoreType.DMA,
    ],
)(_edge_body)


def _node_body(s0_ref, s1_ref, cnt_ref, tgt_ref, wxt_ref, wa0_ref, wa1_ref,
               b_ref, g_ref, bt_ref, out_ref):
    cnt = jnp.maximum(cnt_ref[:, 0:1] + cnt_ref[:, 1:2], 1.0)
    y = lax.dot_general(
        tgt_ref[...], wxt_ref[...], (((1,), (0,)), ((), ())),
        precision=lax.Precision.HIGHEST, preferred_element_type=jnp.float32)
    y += lax.dot_general(
        s0_ref[...] / cnt, wa0_ref[...], (((1,), (0,)), ((), ())),
        precision=lax.Precision.HIGHEST, preferred_element_type=jnp.float32)
    y += lax.dot_general(
        s1_ref[...] / cnt, wa1_ref[...], (((1,), (0,)), ((), ())),
        precision=lax.Precision.HIGHEST, preferred_element_type=jnp.float32)
    y += b_ref[...]
    y = jnp.where(y > 0, y, jnp.exp(y) - 1.0)
    mean = jnp.mean(y, axis=-1, keepdims=True)
    yc = y - mean
    var = jnp.mean(yc * yc, axis=-1, keepdims=True)
    out_ref[...] = yc * lax.rsqrt(var + 1e-5) * g_ref[...] + bt_ref[...]


def _node_finalize(s0, s1, cnt_t, tgt, wxt, wa0, wa1, b, g, bt):
    bn = 2000
    grid = (N // bn,)
    return pl.pallas_call(
        _node_body,
        grid=grid,
        in_specs=[
            pl.BlockSpec((bn, HD), lambda i: (i, 0)),
            pl.BlockSpec((bn, HD), lambda i: (i, 0)),
            pl.BlockSpec((bn, NC), lambda i: (i, 0)),
            pl.BlockSpec((bn, D), lambda i: (i, 0)),
            pl.BlockSpec((D, OUT), lambda i: (0, 0)),
            pl.BlockSpec((HD, OUT), lambda i: (0, 0)),
            pl.BlockSpec((HD, OUT), lambda i: (0, 0)),
            pl.BlockSpec((1, OUT), lambda i: (0, 0)),
            pl.BlockSpec((1, OUT), lambda i: (0, 0)),
            pl.BlockSpec((1, OUT), lambda i: (0, 0)),
        ],
        out_specs=pl.BlockSpec((bn, OUT), lambda i: (i, 0)),
        out_shape=jax.ShapeDtypeStruct((N, OUT), jnp.float32),
    )(s0, s1, cnt_t, tgt, wxt, wa0, wa1, b, g, bt)


def kernel(source_node, target_node, edge_index, edge_attr, distance,
           W_msg, b_msg, W_res, W_comb, b_comb, ln_gamma, ln_beta):
    del edge_attr
    wst = W_msg[:, :D].T
    wtt = W_msg[:, D:2 * D].T
    wd = W_msg[:, 2 * D]
    ps, pt = _project_tables(source_node, target_node, wst, wtt,
                             b_msg.reshape(1, H))
    nck = E // C
    dist_bits = jnp.floor(distance.reshape(E) * 16777216.0).astype(jnp.int32)
    pk = jnp.stack([
        jnp.pad(edge_index[0].reshape(nck, C), ((0, 0), (0, PKW - C))),
        jnp.pad(edge_index[1].reshape(nck, C), ((0, 0), (0, PKW - C))),
        jnp.pad(dist_bits.reshape(nck, C), ((0, 0), (0, PKW - C))),
        jnp.zeros((nck, PKW), jnp.int32),
    ], axis=1)
    sums, cnt_packed = _edge_kernel(ps, pt, pk, wd)
    s0 = sums[0].reshape(2 * AROWS, HD)
    s1 = sums[1].reshape(2 * AROWS, HD)
    cnt_t = cnt_packed[:, :, :P].reshape(NC, CROWS * P).T
    wxt = (W_res + W_comb[:, :D]).T
    wat = W_comb[:, D:].T
    return _node_finalize(s0, s1, cnt_t, target_node, wxt,
                          wat[:HD], wat[HD:],
                          b_comb.reshape(1, OUT),
                          ln_gamma.reshape(1, OUT), ln_beta.reshape(1, OUT))
